# baseline (device time: 91482 ns/iter reference)
import jax
import jax.numpy as jnp
from jax import lax
from jax.experimental import pallas as pl
from jax.experimental.pallas import tpu as pltpu

N_DEV = 4


def kernel(x, w_mat):
    m_full, k_per = x.shape
    k_full, n = w_mat.shape
    blk = m_full // N_DEV

    x = x.astype(jnp.bfloat16)
    w_mat = w_mat.astype(jnp.bfloat16)

    def body(x_ref, w_ref, out_ref, comm_ref, send_sems, recv_sems):
        my = lax.axis_index("i")

        barrier_sem = pltpu.get_barrier_semaphore()
        for d in range(1, N_DEV):
            peer = (my + d) % N_DEV
            pl.semaphore_signal(
                barrier_sem, inc=1,
                device_id=(peer,), device_id_type=pl.DeviceIdType.MESH,
            )
        pl.semaphore_wait(barrier_sem, N_DEV - 1)

        rdmas = []
        for d in range(1, N_DEV):
            dst = (my + d) % N_DEV
            rdma = pltpu.make_async_remote_copy(
                src_ref=x_ref.at[pl.ds(dst * blk, blk), :],
                dst_ref=comm_ref.at[d - 1],
                send_sem=send_sems.at[d - 1],
                recv_sem=recv_sems.at[d - 1],
                device_id=(dst,),
                device_id_type=pl.DeviceIdType.MESH,
            )
            rdma.start()
            rdmas.append(rdma)

        out_ref[...] = jnp.dot(
            x_ref[pl.ds(my * blk, blk), :],
            w_ref[pl.ds(my * blk, blk), :],
            preferred_element_type=jnp.float32,
        )

        for d in range(1, N_DEV):
            rdmas[d - 1].wait_recv()
            src = (my - d) % N_DEV
            out_ref[...] += jnp.dot(
                comm_ref[d - 1],
                w_ref[pl.ds(src * blk, blk), :],
                preferred_element_type=jnp.float32,
            )

        for rdma in rdmas:
            rdma.wait_send()

    return pl.pallas_call(
        body,
        out_shape=jax.ShapeDtypeStruct((blk, n), jnp.float32),
        in_specs=[
            pl.BlockSpec(memory_space=pltpu.VMEM),
            pl.BlockSpec(memory_space=pltpu.VMEM),
        ],
        out_specs=pl.BlockSpec(memory_space=pltpu.VMEM),
        scratch_shapes=[
            pltpu.VMEM((N_DEV - 1, blk, k_per), jnp.bfloat16),
            pltpu.SemaphoreType.DMA((N_DEV - 1,)),
            pltpu.SemaphoreType.DMA((N_DEV - 1,)),
        ],
        compiler_params=pltpu.CompilerParams(collective_id=0),
    )(x, w_mat)


# device time: 69415 ns/iter; 1.3179x vs baseline; 1.3179x over previous
import jax
import jax.numpy as jnp
from jax import lax
from jax.experimental import pallas as pl
from jax.experimental.pallas import tpu as pltpu

N_DEV = 4


def kernel(x, w_mat):
    m_full, k_per = x.shape
    k_full, n = w_mat.shape
    blk = m_full // N_DEV
    bf16 = jnp.bfloat16

    def body(
        x_hbm, w_hbm, out_ref,
        xstage, xbf, wstage, wbf, comm_ref,
        xsems, wsems, send_sems, recv_sems,
    ):
        my = lax.axis_index("i")
        nb1 = (my + 1) % N_DEV
        nb2 = (my + 2) % N_DEV
        nb3 = (my + 3) % N_DEV

        barrier_sem = pltpu.get_barrier_semaphore()
        for d in range(1, N_DEV):
            pl.semaphore_signal(
                barrier_sem, inc=1,
                device_id=((my + d) % N_DEV,),
                device_id_type=pl.DeviceIdType.MESH,
            )
        pl.semaphore_wait(barrier_sem, N_DEV - 1)

        def xcopy(row, slot):
            return pltpu.make_async_copy(
                x_hbm.at[pl.ds(row * blk, blk), :], xstage.at[slot],
                xsems.at[slot],
            )

        def wcopy(row, slot):
            return pltpu.make_async_copy(
                w_hbm.at[pl.ds(row * blk, blk), :], wstage.at[slot],
                wsems.at[slot],
            )

        def rdma(xbf_slot, dst, d):
            return pltpu.make_async_remote_copy(
                src_ref=xbf.at[xbf_slot],
                dst_ref=comm_ref.at[d - 1],
                send_sem=send_sems.at[d - 1],
                recv_sem=recv_sems.at[d - 1],
                device_id=(dst,),
                device_id_type=pl.DeviceIdType.MESH,
            )

        cp_a = xcopy(nb1, 0)
        cp_a.start()
        cp_b = xcopy(nb3, 1)
        cp_b.start()
        wcp_a = wcopy(my, 0)
        wcp_a.start()

        cp_a.wait()
        xbf[0] = xstage[0].astype(bf16)
        rd1 = rdma(0, nb1, 1)
        rd1.start()

        cp_b.wait()
        xbf[1] = xstage[1].astype(bf16)
        rd3 = rdma(1, nb3, 3)
        rd3.start()

        cp_c = xcopy(my, 0)
        cp_c.start()
        wcp_a.wait()
        wbf[0] = wstage[0].astype(bf16)
        wcp_b = wcopy(nb3, 1)
        wcp_b.start()
        cp_c.wait()
        xbf[2] = xstage[0].astype(bf16)
        cp_d = xcopy(nb2, 1)
        cp_d.start()

        out_ref[...] = jnp.dot(
            xbf[2], wbf[0], preferred_element_type=jnp.float32
        )

        wcp_b.wait()
        wbf[1] = wstage[1].astype(bf16)
        wcp_c = wcopy(nb1, 0)
        wcp_c.start()

        rd1.wait_send()
        rd3.wait_send()
        cp_d.wait()
        xbf[3] = xstage[1].astype(bf16)
        rd2 = rdma(3, nb2, 2)
        rd2.start()

        rd1.wait_recv()
        out_ref[...] += jnp.dot(
            comm_ref[0], wbf[1], preferred_element_type=jnp.float32
        )
        wcp_c.wait()
        wbf[0] = wstage[0].astype(bf16)
        wcp_d = wcopy(nb2, 1)
        wcp_d.start()

        rd3.wait_recv()
        out_ref[...] += jnp.dot(
            comm_ref[2], wbf[0], preferred_element_type=jnp.float32
        )
        wcp_d.wait()
        wbf[1] = wstage[1].astype(bf16)

        rd2.wait_recv()
        out_ref[...] += jnp.dot(
            comm_ref[1], wbf[1], preferred_element_type=jnp.float32
        )
        rd2.wait_send()

    return pl.pallas_call(
        body,
        out_shape=jax.ShapeDtypeStruct((blk, n), jnp.float32),
        in_specs=[
            pl.BlockSpec(memory_space=pltpu.MemorySpace.HBM),
            pl.BlockSpec(memory_space=pltpu.MemorySpace.HBM),
        ],
        out_specs=pl.BlockSpec(memory_space=pltpu.VMEM),
        scratch_shapes=[
            pltpu.VMEM((2, blk, k_per), jnp.float32),
            pltpu.VMEM((4, blk, k_per), bf16),
            pltpu.VMEM((2, blk, n), jnp.float32),
            pltpu.VMEM((2, blk, n), bf16),
            pltpu.VMEM((N_DEV - 1, blk, k_per), bf16),
            pltpu.SemaphoreType.DMA((2,)),
            pltpu.SemaphoreType.DMA((2,)),
            pltpu.SemaphoreType.DMA((N_DEV - 1,)),
            pltpu.SemaphoreType.DMA((N_DEV - 1,)),
        ],
        compiler_params=pltpu.CompilerParams(
            collective_id=0,
            vmem_limit_bytes=60 * 1024 * 1024,
        ),
    )(x, w_mat)


# device time: 65495 ns/iter; 1.3968x vs baseline; 1.0599x over previous
import jax
import jax.numpy as jnp
from jax import lax
from jax.experimental import pallas as pl
from jax.experimental.pallas import tpu as pltpu

N_DEV = 4


def kernel(x, w_mat):
    m_full, k_per = x.shape
    k_full, n = w_mat.shape
    blk = m_full // N_DEV
    h = blk // 2
    bf16 = jnp.bfloat16
    f32 = jnp.float32

    def body(
        x_hbm, w_hbm, out_hbm,
        acc, xstage, xbf, wstage, wbf, comm_ref,
        xsems, wsems, osems, send_sems, recv_sems,
    ):
        my = lax.axis_index("i")
        nb1 = (my + 1) % N_DEV
        nb2 = (my + 2) % N_DEV
        nb3 = (my + 3) % N_DEV

        barrier_sem = pltpu.get_barrier_semaphore()
        for d in range(1, N_DEV):
            pl.semaphore_signal(
                barrier_sem, inc=1,
                device_id=((my + d) % N_DEV,),
                device_id_type=pl.DeviceIdType.MESH,
            )
        pl.semaphore_wait(barrier_sem, N_DEV - 1)

        def xcopy(row, slot):
            return pltpu.make_async_copy(
                x_hbm.at[pl.ds(row * blk, blk), :], xstage.at[slot],
                xsems.at[slot],
            )

        def wcopy(row, slot):
            return pltpu.make_async_copy(
                w_hbm.at[pl.ds(row * blk, blk), :], wstage.at[slot],
                wsems.at[slot],
            )

        def rdma(xbf_slot, dst, d, c):
            return pltpu.make_async_remote_copy(
                src_ref=xbf.at[xbf_slot, pl.ds(c * h, h), :],
                dst_ref=comm_ref.at[d - 1, pl.ds(c * h, h), :],
                send_sem=send_sems.at[d - 1, c],
                recv_sem=recv_sems.at[d - 1, c],
                device_id=(dst,),
                device_id_type=pl.DeviceIdType.MESH,
            )

        cp_a = xcopy(nb1, 0)
        cp_a.start()
        cp_b = xcopy(nb3, 1)
        cp_b.start()
        wcp_a = wcopy(my, 0)
        wcp_a.start()

        cp_a.wait()
        xbf[0] = xstage[0].astype(bf16)
        rd1 = [rdma(0, nb1, 1, c) for c in range(2)]
        rd1[0].start()
        rd1[1].start()

        cp_b.wait()
        xbf[1] = xstage[1].astype(bf16)
        rd3 = [rdma(1, nb3, 3, c) for c in range(2)]
        rd3[0].start()
        rd3[1].start()

        cp_c = xcopy(my, 0)
        cp_c.start()
        wcp_a.wait()
        wbf[0] = wstage[0].astype(bf16)
        wcp_b = wcopy(nb3, 1)
        wcp_b.start()
        cp_c.wait()
        xbf[2] = xstage[0].astype(bf16)
        cp_d = xcopy(nb2, 1)
        cp_d.start()
        wcp_b.wait()
        wbf[1] = wstage[1].astype(bf16)
        wcp_c = wcopy(nb1, 0)
        wcp_c.start()

        acc[...] = jnp.dot(xbf[2], wbf[0], preferred_element_type=f32)

        wcp_c.wait()
        wbf[0] = wstage[0].astype(bf16)
        wcp_d = wcopy(nb2, 1)
        wcp_d.start()

        rd1[0].wait_recv()
        acc[pl.ds(0, h), :] += jnp.dot(
            comm_ref[0, pl.ds(0, h), :], wbf[1], preferred_element_type=f32
        )
        rd3[0].wait_recv()
        acc[pl.ds(0, h), :] += jnp.dot(
            comm_ref[2, pl.ds(0, h), :], wbf[0], preferred_element_type=f32
        )

        for r in rd1 + rd3:
            r.wait_send()
        cp_d.wait()
        xbf[3] = xstage[1].astype(bf16)
        rd2 = [rdma(3, nb2, 2, c) for c in range(2)]
        rd2[0].start()
        rd2[1].start()

        rd1[1].wait_recv()
        acc[pl.ds(h, h), :] += jnp.dot(
            comm_ref[0, pl.ds(h, h), :], wbf[1], preferred_element_type=f32
        )
        rd3[1].wait_recv()
        acc[pl.ds(h, h), :] += jnp.dot(
            comm_ref[2, pl.ds(h, h), :], wbf[0], preferred_element_type=f32
        )

        wcp_d.wait()
        wbf[1] = wstage[1].astype(bf16)

        rd2[0].wait_recv()
        acc[pl.ds(0, h), :] += jnp.dot(
            comm_ref[1, pl.ds(0, h), :], wbf[1], preferred_element_type=f32
        )
        od0 = pltpu.make_async_copy(
            acc.at[pl.ds(0, h), :], out_hbm.at[pl.ds(0, h), :], osems.at[0]
        )
        od0.start()

        rd2[1].wait_recv()
        acc[pl.ds(h, h), :] += jnp.dot(
            comm_ref[1, pl.ds(h, h), :], wbf[1], preferred_element_type=f32
        )
        od1 = pltpu.make_async_copy(
            acc.at[pl.ds(h, h), :], out_hbm.at[pl.ds(h, h), :], osems.at[1]
        )
        od1.start()

        od0.wait()
        od1.wait()
        rd2[0].wait_send()
        rd2[1].wait_send()

    return pl.pallas_call(
        body,
        out_shape=jax.ShapeDtypeStruct((blk, n), f32),
        in_specs=[
            pl.BlockSpec(memory_space=pltpu.MemorySpace.HBM),
            pl.BlockSpec(memory_space=pltpu.MemorySpace.HBM),
        ],
        out_specs=pl.BlockSpec(memory_space=pltpu.MemorySpace.HBM),
        scratch_shapes=[
            pltpu.VMEM((blk, n), f32),
            pltpu.VMEM((2, blk, k_per), f32),
            pltpu.VMEM((4, blk, k_per), bf16),
            pltpu.VMEM((2, blk, n), f32),
            pltpu.VMEM((2, blk, n), bf16),
            pltpu.VMEM((N_DEV - 1, blk, k_per), bf16),
            pltpu.SemaphoreType.DMA((2,)),
            pltpu.SemaphoreType.DMA((2,)),
            pltpu.SemaphoreType.DMA((2,)),
            pltpu.SemaphoreType.DMA((N_DEV - 1, 2)),
            pltpu.SemaphoreType.DMA((N_DEV - 1, 2)),
        ],
        compiler_params=pltpu.CompilerParams(
            collective_id=0,
            vmem_limit_bytes=63 * 1024 * 1024,
        ),
    )(x, w_mat)


# device time: 62678 ns/iter; 1.4596x vs baseline; 1.0449x over previous
import jax
import jax.numpy as jnp
from jax import lax
from jax.experimental import pallas as pl
from jax.experimental.pallas import tpu as pltpu

N_DEV = 4


def kernel(x, w_mat):
    m_full, k_per = x.shape
    k_full, n = w_mat.shape
    blk = m_full // N_DEV
    h = blk // 2
    dh = 3 * blk // 4
    bf16 = jnp.bfloat16
    f32 = jnp.float32

    def body(
        x_hbm, w_hbm, out_hbm,
        acc, xstage, xbf, wstage, wbf, comm_ref,
        xsems, wsems, osems, send_sems, recv_sems,
    ):
        my = lax.axis_index("i")
        nb1 = (my + 1) % N_DEV
        nb2 = (my + 2) % N_DEV
        nb3 = (my + 3) % N_DEV

        barrier_sem = pltpu.get_barrier_semaphore()
        for d in range(1, N_DEV):
            pl.semaphore_signal(
                barrier_sem, inc=1,
                device_id=((my + d) % N_DEV,),
                device_id_type=pl.DeviceIdType.MESH,
            )
        pl.semaphore_wait(barrier_sem, N_DEV - 1)

        def xcopy(row0, nrows, slot, roff, sem):
            return pltpu.make_async_copy(
                x_hbm.at[pl.ds(row0, nrows), :],
                xstage.at[slot, pl.ds(roff, nrows), :],
                xsems.at[sem],
            )

        def wcopy(row, slot):
            return pltpu.make_async_copy(
                w_hbm.at[pl.ds(row * blk, blk), :], wstage.at[slot],
                wsems.at[slot],
            )

        def rdma(xbf_slot, dst, d, roff, nrows, c):
            return pltpu.make_async_remote_copy(
                src_ref=xbf.at[xbf_slot, pl.ds(roff, nrows), :],
                dst_ref=comm_ref.at[d - 1, pl.ds(roff, nrows), :],
                send_sem=send_sems.at[d - 1, c],
                recv_sem=recv_sems.at[d - 1, c],
                device_id=(dst,),
                device_id_type=pl.DeviceIdType.MESH,
            )

        cp_a0 = xcopy(nb1 * blk, h, 0, 0, 0)
        cp_a0.start()
        cp_a1 = xcopy(nb1 * blk + h, h, 0, h, 1)
        cp_a1.start()
        cp_b0 = xcopy(nb3 * blk, h, 1, 0, 2)
        cp_b0.start()
        cp_b1 = xcopy(nb3 * blk + h, h, 1, h, 3)
        cp_b1.start()
        wcp_a = wcopy(my, 0)
        wcp_a.start()

        cp_a0.wait()
        xbf[0, pl.ds(0, h), :] = xstage[0, pl.ds(0, h), :].astype(bf16)
        rd1 = [rdma(0, nb1, 1, 0, h, 0), rdma(0, nb1, 1, h, h, 1)]
        rd1[0].start()
        cp_b0.wait()
        xbf[1, pl.ds(0, h), :] = xstage[1, pl.ds(0, h), :].astype(bf16)
        rd3 = [rdma(1, nb3, 3, 0, h, 0), rdma(1, nb3, 3, h, h, 1)]
        rd3[0].start()
        cp_a1.wait()
        xbf[0, pl.ds(h, h), :] = xstage[0, pl.ds(h, h), :].astype(bf16)
        rd1[1].start()
        cp_b1.wait()
        xbf[1, pl.ds(h, h), :] = xstage[1, pl.ds(h, h), :].astype(bf16)
        rd3[1].start()

        cp_c = xcopy(my * blk, blk, 0, 0, 0)
        cp_c.start()
        wcp_a.wait()
        wbf[0] = wstage[0].astype(bf16)
        wcp_b = wcopy(nb3, 1)
        wcp_b.start()
        cp_c.wait()
        xbf[2] = xstage[0].astype(bf16)
        cp_d = xcopy(nb2 * blk, blk, 1, 0, 1)
        cp_d.start()
        wcp_b.wait()
        wbf[1] = wstage[1].astype(bf16)
        wcp_c = wcopy(nb1, 0)
        wcp_c.start()
        cp_d.wait()
        xbf[3] = xstage[1].astype(bf16)

        acc[...] = jnp.dot(xbf[2], wbf[0], preferred_element_type=f32)

        wcp_c.wait()
        wbf[0] = wstage[0].astype(bf16)
        wcp_d = wcopy(nb2, 1)
        wcp_d.start()

        rd1[0].wait_recv()
        acc[pl.ds(0, h), :] += jnp.dot(
            comm_ref[0, pl.ds(0, h), :], wbf[1], preferred_element_type=f32
        )
        rd3[0].wait_recv()
        acc[pl.ds(0, h), :] += jnp.dot(
            comm_ref[2, pl.ds(0, h), :], wbf[0], preferred_element_type=f32
        )

        for r in rd1 + rd3:
            r.wait_send()
        rd2 = [
            rdma(3, nb2, 2, 0, dh, 0),
            rdma(3, nb2, 2, dh, blk - dh, 1),
        ]
        rd2[0].start()
        rd2[1].start()

        rd1[1].wait_recv()
        acc[pl.ds(h, h), :] += jnp.dot(
            comm_ref[0, pl.ds(h, h), :], wbf[1], preferred_element_type=f32
        )
        rd3[1].wait_recv()
        acc[pl.ds(h, h), :] += jnp.dot(
            comm_ref[2, pl.ds(h, h), :], wbf[0], preferred_element_type=f32
        )

        wcp_d.wait()
        wbf[1] = wstage[1].astype(bf16)

        rd2[0].wait_recv()
        acc[pl.ds(0, dh), :] += jnp.dot(
            comm_ref[1, pl.ds(0, dh), :], wbf[1], preferred_element_type=f32
        )
        od0 = pltpu.make_async_copy(
            acc.at[pl.ds(0, dh), :], out_hbm.at[pl.ds(0, dh), :], osems.at[0]
        )
        od0.start()

        rd2[1].wait_recv()
        acc[pl.ds(dh, blk - dh), :] += jnp.dot(
            comm_ref[1, pl.ds(dh, blk - dh), :], wbf[1],
            preferred_element_type=f32,
        )
        od1 = pltpu.make_async_copy(
            acc.at[pl.ds(dh, blk - dh), :],
            out_hbm.at[pl.ds(dh, blk - dh), :],
            osems.at[1],
        )
        od1.start()

        od0.wait()
        od1.wait()
        rd2[0].wait_send()
        rd2[1].wait_send()

    return pl.pallas_call(
        body,
        out_shape=jax.ShapeDtypeStruct((blk, n), f32),
        in_specs=[
            pl.BlockSpec(memory_space=pltpu.MemorySpace.HBM),
            pl.BlockSpec(memory_space=pltpu.MemorySpace.HBM),
        ],
        out_specs=pl.BlockSpec(memory_space=pltpu.MemorySpace.HBM),
        scratch_shapes=[
            pltpu.VMEM((blk, n), f32),
            pltpu.VMEM((2, blk, k_per), f32),
            pltpu.VMEM((4, blk, k_per), bf16),
            pltpu.VMEM((2, blk, n), f32),
            pltpu.VMEM((2, blk, n), bf16),
            pltpu.VMEM((N_DEV - 1, blk, k_per), bf16),
            pltpu.SemaphoreType.DMA((4,)),
            pltpu.SemaphoreType.DMA((2,)),
            pltpu.SemaphoreType.DMA((2,)),
            pltpu.SemaphoreType.DMA((N_DEV - 1, 2)),
            pltpu.SemaphoreType.DMA((N_DEV - 1, 2)),
        ],
        compiler_params=pltpu.CompilerParams(
            collective_id=0,
            vmem_limit_bytes=63 * 1024 * 1024,
        ),
    )(x, w_mat)


# device time: 61704 ns/iter; 1.4826x vs baseline; 1.0158x over previous
import jax
import jax.numpy as jnp
from jax import lax
from jax.experimental import pallas as pl
from jax.experimental.pallas import tpu as pltpu

N_DEV = 4


def kernel(x, w_mat):
    m_full, k_per = x.shape
    k_full, n = w_mat.shape
    blk = m_full // N_DEV
    h = blk // 2
    dh = 3 * blk // 4
    bf16 = jnp.bfloat16
    f32 = jnp.float32

    def body(
        x_hbm, w_hbm, out_hbm,
        acc, xstage, xbf, wstage, wbf, comm_ref,
        xsems, wsems, osems, send_sems, recv_sems,
    ):
        my = lax.axis_index("i")
        nb1 = (my + 1) % N_DEV
        nb2 = (my + 2) % N_DEV
        nb3 = (my + 3) % N_DEV

        def xcopy(row0, nrows, slot, roff, sem):
            return pltpu.make_async_copy(
                x_hbm.at[pl.ds(row0, nrows), :],
                xstage.at[slot, pl.ds(roff, nrows), :],
                xsems.at[sem],
            )

        def wcopy(row, slot):
            return pltpu.make_async_copy(
                w_hbm.at[pl.ds(row * blk, blk), :], wstage.at[slot],
                wsems.at[slot],
            )

        def rdma(xbf_slot, dst, d, roff, nrows, c):
            return pltpu.make_async_remote_copy(
                src_ref=xbf.at[xbf_slot, pl.ds(roff, nrows), :],
                dst_ref=comm_ref.at[d - 1, pl.ds(roff, nrows), :],
                send_sem=send_sems.at[d - 1, c],
                recv_sem=recv_sems.at[d - 1, c],
                device_id=(dst,),
                device_id_type=pl.DeviceIdType.MESH,
            )

        cp_a0 = xcopy(nb1 * blk, h, 0, 0, 0)
        cp_a0.start()
        cp_b0 = xcopy(nb3 * blk, h, 1, 0, 2)
        cp_b0.start()
        cp_a1 = xcopy(nb1 * blk + h, h, 0, h, 1)
        cp_a1.start()
        cp_b1 = xcopy(nb3 * blk + h, h, 1, h, 3)
        cp_b1.start()
        wcp_a = wcopy(my, 0)
        wcp_a.start()

        barrier_sem = pltpu.get_barrier_semaphore()
        for d in range(1, N_DEV):
            pl.semaphore_signal(
                barrier_sem, inc=1,
                device_id=((my + d) % N_DEV,),
                device_id_type=pl.DeviceIdType.MESH,
            )
        pl.semaphore_wait(barrier_sem, N_DEV - 1)

        cp_a0.wait()
        xbf[0, pl.ds(0, h), :] = xstage[0, pl.ds(0, h), :].astype(bf16)
        rd1 = [rdma(0, nb1, 1, 0, h, 0), rdma(0, nb1, 1, h, h, 1)]
        rd1[0].start()
        cp_b0.wait()
        xbf[1, pl.ds(0, h), :] = xstage[1, pl.ds(0, h), :].astype(bf16)
        rd3 = [rdma(1, nb3, 3, 0, h, 0), rdma(1, nb3, 3, h, h, 1)]
        rd3[0].start()
        cp_a1.wait()
        xbf[0, pl.ds(h, h), :] = xstage[0, pl.ds(h, h), :].astype(bf16)
        rd1[1].start()
        cp_b1.wait()
        xbf[1, pl.ds(h, h), :] = xstage[1, pl.ds(h, h), :].astype(bf16)
        rd3[1].start()

        cp_c = xcopy(my * blk, blk, 0, 0, 0)
        cp_c.start()
        wcp_a.wait()
        wbf[0] = wstage[0].astype(bf16)
        wcp_b = wcopy(nb3, 1)
        wcp_b.start()
        cp_c.wait()
        xbf[2] = xstage[0].astype(bf16)
        cp_d = xcopy(nb2 * blk, blk, 1, 0, 1)
        cp_d.start()
        wcp_b.wait()
        wbf[1] = wstage[1].astype(bf16)
        wcp_c = wcopy(nb1, 0)
        wcp_c.start()
        cp_d.wait()
        xbf[3] = xstage[1].astype(bf16)

        acc[...] = jnp.dot(xbf[2], wbf[0], preferred_element_type=f32)

        wcp_c.wait()
        wbf[0] = wstage[0].astype(bf16)
        wcp_d = wcopy(nb2, 1)
        wcp_d.start()

        rd1[0].wait_recv()
        acc[pl.ds(0, h), :] += jnp.dot(
            comm_ref[0, pl.ds(0, h), :], wbf[1], preferred_element_type=f32
        )
        rd3[0].wait_recv()
        acc[pl.ds(0, h), :] += jnp.dot(
            comm_ref[2, pl.ds(0, h), :], wbf[0], preferred_element_type=f32
        )

        for r in rd1 + rd3:
            r.wait_send()
        rd2 = [
            rdma(3, nb2, 2, 0, dh, 0),
            rdma(3, nb2, 2, dh, blk - dh, 1),
        ]
        rd2[0].start()
        rd2[1].start()

        rd1[1].wait_recv()
        acc[pl.ds(h, h), :] += jnp.dot(
            comm_ref[0, pl.ds(h, h), :], wbf[1], preferred_element_type=f32
        )
        rd3[1].wait_recv()
        acc[pl.ds(h, h), :] += jnp.dot(
            comm_ref[2, pl.ds(h, h), :], wbf[0], preferred_element_type=f32
        )

        wcp_d.wait()
        wbf[1] = wstage[1].astype(bf16)

        rd2[0].wait_recv()
        acc[pl.ds(0, dh), :] += jnp.dot(
            comm_ref[1, pl.ds(0, dh), :], wbf[1], preferred_element_type=f32
        )
        od0 = pltpu.make_async_copy(
            acc.at[pl.ds(0, dh), :], out_hbm.at[pl.ds(0, dh), :], osems.at[0]
        )
        od0.start()

        rd2[1].wait_recv()
        acc[pl.ds(dh, blk - dh), :] += jnp.dot(
            comm_ref[1, pl.ds(dh, blk - dh), :], wbf[1],
            preferred_element_type=f32,
        )
        od1 = pltpu.make_async_copy(
            acc.at[pl.ds(dh, blk - dh), :],
            out_hbm.at[pl.ds(dh, blk - dh), :],
            osems.at[1],
        )
        od1.start()

        od0.wait()
        od1.wait()
        rd2[0].wait_send()
        rd2[1].wait_send()

    return pl.pallas_call(
        body,
        out_shape=jax.ShapeDtypeStruct((blk, n), f32),
        in_specs=[
            pl.BlockSpec(memory_space=pltpu.MemorySpace.HBM),
            pl.BlockSpec(memory_space=pltpu.MemorySpace.HBM),
        ],
        out_specs=pl.BlockSpec(memory_space=pltpu.MemorySpace.HBM),
        scratch_shapes=[
            pltpu.VMEM((blk, n), f32),
            pltpu.VMEM((2, blk, k_per), f32),
            pltpu.VMEM((4, blk, k_per), bf16),
            pltpu.VMEM((2, blk, n), f32),
            pltpu.VMEM((2, blk, n), bf16),
            pltpu.VMEM((N_DEV - 1, blk, k_per), bf16),
            pltpu.SemaphoreType.DMA((4,)),
            pltpu.SemaphoreType.DMA((2,)),
            pltpu.SemaphoreType.DMA((2,)),
            pltpu.SemaphoreType.DMA((N_DEV - 1, 2)),
            pltpu.SemaphoreType.DMA((N_DEV - 1, 2)),
        ],
        compiler_params=pltpu.CompilerParams(
            collective_id=0,
            vmem_limit_bytes=63 * 1024 * 1024,
        ),
    )(x, w_mat)
